# baseline (speedup 1.0000x reference)
import functools

import jax
import jax.numpy as jnp
from jax.experimental import pallas as pl
from jax.experimental.pallas import tpu as pltpu

K_IN = 900
OUT_W = 98
PAD_W = 128
BN_EPS = 1e-5
SP_THR = 20.0
LN2 = 0.6931471805599453


def _fused(x_ref, w_ref, g_ref, b_ref, o_ref,
           conv_buf, s1, s2, scale, shift, *, n, num_tiles):
    p = pl.program_id(0)
    i = pl.program_id(1)

    @pl.when((p == 0) & (i == 0))
    def _init():
        s1[...] = jnp.zeros_like(s1)
        s2[...] = jnp.zeros_like(s2)

    @pl.when(p == 0)
    def _conv_stats():
        conv = jnp.dot(x_ref[...], w_ref[...],
                       preferred_element_type=jnp.float32)
        conv_buf[i] = conv
        s1[...] += jnp.sum(conv, axis=0, keepdims=True)
        s2[...] += jnp.sum(conv * conv, axis=0, keepdims=True)

    @pl.when((p == 0) & (i == num_tiles - 1))
    def _finalize():
        inv_n = jnp.float32(1.0 / n)
        mean = s1[...] * inv_n
        var = jnp.maximum(s2[...] * inv_n - mean * mean, 0.0)
        sc = g_ref[...] * jax.lax.rsqrt(var + BN_EPS)
        scale[...] = sc
        shift[...] = b_ref[...] - mean * sc

    @pl.when(p == 1)
    def _bn_softplus():
        y = conv_buf[i] * scale[...] + shift[...]
        sp = jnp.log1p(jnp.exp(jnp.minimum(y, SP_THR)))
        o_ref[...] = jnp.where(y > SP_THR, y, sp)[:, :OUT_W]


@jax.jit
def kernel(x, wmat, gamma, beta):
    n = x.shape[0]
    tile_n = 1024 if n % 1024 == 0 else 8
    num_tiles = n // tile_n

    x_flat = x.reshape(n, K_IN)
    g_p = jnp.zeros((1, PAD_W), jnp.float32).at[0, :OUT_W].set(
        gamma.astype(jnp.float32).reshape(-1))
    b_p = jnp.zeros((1, PAD_W), jnp.float32).at[0, :OUT_W].set(
        beta.astype(jnp.float32).reshape(-1))

    return pl.pallas_call(
        functools.partial(_fused, n=n, num_tiles=num_tiles),
        out_shape=jax.ShapeDtypeStruct((n, OUT_W), jnp.float32),
        grid=(2, num_tiles),
        in_specs=[
            pl.BlockSpec((tile_n, K_IN),
                         lambda p, i: (i * (1 - p) + (num_tiles - 1) * p, 0)),
            pl.BlockSpec((K_IN, PAD_W), lambda p, i: (0, 0)),
            pl.BlockSpec((1, PAD_W), lambda p, i: (0, 0)),
            pl.BlockSpec((1, PAD_W), lambda p, i: (0, 0)),
        ],
        out_specs=pl.BlockSpec((tile_n, OUT_W), lambda p, i: (i * p, 0)),
        scratch_shapes=[
            pltpu.VMEM((num_tiles, tile_n, PAD_W), jnp.float32),
            pltpu.VMEM((1, PAD_W), jnp.float32),
            pltpu.VMEM((1, PAD_W), jnp.float32),
            pltpu.VMEM((1, PAD_W), jnp.float32),
            pltpu.VMEM((1, PAD_W), jnp.float32),
        ],
        compiler_params=pltpu.CompilerParams(
            dimension_semantics=("arbitrary", "arbitrary"),
            vmem_limit_bytes=60 * 1024 * 1024,
        ),
    )(x_flat, wmat, g_p, b_p)
